# baseline (speedup 1.0000x reference)
import functools

import jax
import jax.numpy as jnp
from jax import lax
from jax.experimental import pallas as pl
from jax.experimental.pallas import tpu as pltpu
from jax.experimental.pallas import tpu_sc as plsc

NC = 2
NS = 16
L = 16
NW = NC * NS

CH = 128


def _floor_f32(x):
    t = x.astype(jnp.int32)
    tf = t.astype(jnp.float32)
    ti = jnp.where(tf > x, t - 1, t)
    return ti, ti.astype(jnp.float32)


def _make_cast_bf16(N, C, H, W):
    def body(in_ref, out_ref):
        out_ref[...] = in_ref[...].astype(jnp.bfloat16)

    YB = 16
    return pl.pallas_call(
        body,
        grid=(N, H // YB),
        in_specs=[pl.BlockSpec((1, C, YB, W), lambda n, y: (n, 0, y, 0))],
        out_specs=pl.BlockSpec((1, C, YB, W), lambda n, y: (n, 0, y, 0)),
        out_shape=jax.ShapeDtypeStruct((N, C, H, W), jnp.bfloat16),
    )


def _make_sc_kernel(N, C, H, W, Ho, Wo):
    P = N * Ho * Wo
    HW = H * W
    assert P % NW == 0
    ppw = P // NW
    assert ppw % CH == 0
    nchunk = ppw // CH
    assert nchunk % 2 == 0
    cg2 = C // (2 * L)

    mesh = plsc.VectorSubcoreMesh(
        core_axis_name="c", subcore_axis_name="s", num_cores=NC,
        num_subcores=NS)

    @functools.partial(
        pl.kernel,
        out_type=jax.ShapeDtypeStruct((P, C), jnp.bfloat16),
        mesh=mesh,
        scratch_types=[
            pltpu.VMEM((ppw,), jnp.float32),
            pltpu.VMEM((ppw,), jnp.float32),
            [pltpu.VMEM((4, CH), jnp.int32) for _ in range(2)],
            [pltpu.VMEM((4, CH), jnp.float32) for _ in range(2)],
            [[pltpu.VMEM((CH, C), jnp.bfloat16) for _ in range(4)]
             for _ in range(2)],
            pltpu.VMEM((CH, C), jnp.bfloat16),
            [pltpu.SemaphoreType.DMA for _ in range(2)],
        ],
        compiler_params=pltpu.CompilerParams(
            use_tc_tiling_on_sc=False, needs_layout_passes=False,
            skip_device_barrier=True),
    )
    def grid_sample_sc(gx_hbm, gy_hbm, table_hbm, out_hbm,
                       gx_v, gy_v, idx_v, w_v, rows_v, out_v, sems):
        wid = lax.axis_index("s") * NC + lax.axis_index("c")
        base_w = wid * ppw
        nbase = (base_w // (Ho * Wo)) * HW

        pltpu.sync_copy(gx_hbm.at[pl.ds(base_w, ppw)], gx_v)
        pltpu.sync_copy(gy_hbm.at[pl.ds(base_w, ppw)], gy_v)

        def stage_chunk(c, b):
            for g in range(CH // L):
                sl = pl.ds(c * CH + g * L, L)
                osl = pl.ds(g * L, L)
                gx = gx_v[sl]
                gy = gy_v[sl]
                ix = (gx + 1.0) * (W * 0.5) - 0.5
                iy = (gy + 1.0) * (H * 0.5) - 0.5
                ix0i, ix0f = _floor_f32(ix)
                iy0i, iy0f = _floor_f32(iy)
                wx1 = ix - ix0f
                wy1 = iy - iy0f

                mx0 = (ix0f >= 0.0) & (ix0f <= W - 1.0)
                mx1 = (ix0f >= -1.0) & (ix0f <= W - 2.0)
                my0 = (iy0f >= 0.0) & (iy0f <= H - 1.0)
                my1 = (iy0f >= -1.0) & (iy0f <= H - 2.0)
                w_v[b][0, osl] = jnp.where(mx0, 1.0 - wx1, 0.0)
                w_v[b][1, osl] = jnp.where(mx1, wx1, 0.0)
                w_v[b][2, osl] = jnp.where(my0, 1.0 - wy1, 0.0)
                w_v[b][3, osl] = jnp.where(my1, wy1, 0.0)

                xi0 = jnp.clip(ix0i, 0, W - 1)
                xi1 = jnp.clip(ix0i + 1, 0, W - 1)
                yi0 = jnp.clip(iy0i, 0, H - 1) * W + nbase
                yi1 = jnp.clip(iy0i + 1, 0, H - 1) * W + nbase
                idx_v[b][0, osl] = yi0 + xi0
                idx_v[b][1, osl] = yi0 + xi1
                idx_v[b][2, osl] = yi1 + xi0
                idx_v[b][3, osl] = yi1 + xi1

            for q in range(4):
                pltpu.async_copy(table_hbm.at[idx_v[b].at[q]], rows_v[b][q],
                                 sems[b])

        def finish_chunk(c, b):
            for q in range(4):
                pltpu.make_async_copy(table_hbm.at[idx_v[b].at[q]],
                                      rows_v[b][q], sems[b]).wait()

            @pl.loop(0, CH // L)
            def _grp(g):
                sl = pl.ds(g * L, L)
                a0 = w_v[b][0, sl]
                a1 = w_v[b][1, sl]
                b0 = w_v[b][2, sl]
                b1 = w_v[b][3, sl]
                def dup_bf16(v):
                    u = plsc.bitcast(v, jnp.uint32) + jnp.uint32(0x8000)
                    w = (u & jnp.uint32(0xFFFF0000)) | (u >> jnp.uint32(16))
                    return plsc.bitcast(w, jnp.bfloat16)

                for ll in range(L):
                    i = g * L + ll
                    lane = jnp.full((L,), ll, jnp.int32)
                    a0s = dup_bf16(a0.at[lane].get(mode="promise_in_bounds"))
                    a1s = dup_bf16(a1.at[lane].get(mode="promise_in_bounds"))
                    b0s = dup_bf16(b0.at[lane].get(mode="promise_in_bounds"))
                    b1s = dup_bf16(b1.at[lane].get(mode="promise_in_bounds"))
                    for j in range(cg2):
                        cs = pl.ds(j * 2 * L, 2 * L)
                        t0 = a0s * rows_v[b][0][i, cs] + a1s * rows_v[b][1][i, cs]
                        t1 = a0s * rows_v[b][2][i, cs] + a1s * rows_v[b][3][i, cs]
                        out_v[i, cs] = b0s * t0 + b1s * t1

            pltpu.sync_copy(out_v, out_hbm.at[pl.ds(base_w + c * CH, CH)])

        stage_chunk(0, 0)

        @pl.loop(0, nchunk, step=2)
        def _chunk(k):
            for b in range(2):
                c = k + b

                @pl.when(c + 1 < nchunk)
                def _prefetch():
                    stage_chunk(c + 1, (b + 1) % 2)

                finish_chunk(c, b)

    return grid_sample_sc


def kernel(inp, grid):
    N, C, H, W = inp.shape
    _, Ho, Wo, _ = grid.shape
    inp_bf = _make_cast_bf16(N, C, H, W)(inp)
    table = inp_bf.transpose(0, 2, 3, 1).reshape(N * H * W, C)
    gx = grid[..., 0].reshape(-1)
    gy = grid[..., 1].reshape(-1)
    sc = _make_sc_kernel(N, C, H, W, Ho, Wo)
    out_rows = sc(gx, gy, table)
    return (out_rows.astype(jnp.float32)
            .reshape(N, Ho, Wo, C).transpose(0, 3, 1, 2))


# speedup vs baseline: 1.1863x; 1.1863x over previous
import functools

import jax
import jax.numpy as jnp
from jax import lax
from jax.experimental import pallas as pl
from jax.experimental.pallas import tpu as pltpu
from jax.experimental.pallas import tpu_sc as plsc

NC = 2
NS = 16
L = 16
NW = NC * NS

CH = 64


def _floor_f32(x):
    t = x.astype(jnp.int32)
    tf = t.astype(jnp.float32)
    ti = jnp.where(tf > x, t - 1, t)
    return ti, ti.astype(jnp.float32)


def _make_sc_kernel(N, C, H, W, Ho, Wo):
    P = N * Ho * Wo
    HW = H * W
    assert P % NW == 0
    ppw = P // NW
    assert ppw % CH == 0
    nchunk = ppw // CH
    assert nchunk % 2 == 0
    cg = C // L

    mesh = plsc.VectorSubcoreMesh(
        core_axis_name="c", subcore_axis_name="s", num_cores=NC,
        num_subcores=NS)

    @functools.partial(
        pl.kernel,
        out_type=jax.ShapeDtypeStruct((P, C), jnp.float32),
        mesh=mesh,
        scratch_types=[
            pltpu.VMEM((ppw,), jnp.float32),
            pltpu.VMEM((ppw,), jnp.float32),
            [pltpu.VMEM((4, CH), jnp.int32) for _ in range(2)],
            [pltpu.VMEM((4, CH), jnp.float32) for _ in range(2)],
            [[pltpu.VMEM((CH, C), jnp.float32) for _ in range(4)]
             for _ in range(2)],
            [pltpu.VMEM((CH, C), jnp.float32) for _ in range(2)],
            [pltpu.SemaphoreType.DMA for _ in range(2)],
            [pltpu.SemaphoreType.DMA for _ in range(2)],
        ],
        compiler_params=pltpu.CompilerParams(
            use_tc_tiling_on_sc=False, skip_device_barrier=True),
    )
    def grid_sample_sc(gx_hbm, gy_hbm, table_hbm, out_hbm,
                       gx_v, gy_v, idx_v, w_v, rows_v, out_v, sems, semo):
        wid = lax.axis_index("s") * NC + lax.axis_index("c")
        base_w = wid * ppw
        nbase = (base_w // (Ho * Wo)) * HW

        pltpu.sync_copy(gx_hbm.at[pl.ds(base_w, ppw)], gx_v)
        pltpu.sync_copy(gy_hbm.at[pl.ds(base_w, ppw)], gy_v)

        def stage_chunk(c, b):
            for g in range(CH // L):
                sl = pl.ds(c * CH + g * L, L)
                osl = pl.ds(g * L, L)
                gx = gx_v[sl]
                gy = gy_v[sl]
                ix = (gx + 1.0) * (W * 0.5) - 0.5
                iy = (gy + 1.0) * (H * 0.5) - 0.5
                ix0i, ix0f = _floor_f32(ix)
                iy0i, iy0f = _floor_f32(iy)
                wx1 = ix - ix0f
                wy1 = iy - iy0f

                mx0 = (ix0f >= 0.0) & (ix0f <= W - 1.0)
                mx1 = (ix0f >= -1.0) & (ix0f <= W - 2.0)
                my0 = (iy0f >= 0.0) & (iy0f <= H - 1.0)
                my1 = (iy0f >= -1.0) & (iy0f <= H - 2.0)
                w_v[b][0, osl] = jnp.where(mx0, 1.0 - wx1, 0.0)
                w_v[b][1, osl] = jnp.where(mx1, wx1, 0.0)
                w_v[b][2, osl] = jnp.where(my0, 1.0 - wy1, 0.0)
                w_v[b][3, osl] = jnp.where(my1, wy1, 0.0)

                xi0 = jnp.clip(ix0i, 0, W - 1)
                xi1 = jnp.clip(ix0i + 1, 0, W - 1)
                yi0 = jnp.clip(iy0i, 0, H - 1) * W + nbase
                yi1 = jnp.clip(iy0i + 1, 0, H - 1) * W + nbase
                idx_v[b][0, osl] = yi0 + xi0
                idx_v[b][1, osl] = yi0 + xi1
                idx_v[b][2, osl] = yi1 + xi0
                idx_v[b][3, osl] = yi1 + xi1

            for q in range(4):
                pltpu.async_copy(table_hbm.at[idx_v[b].at[q]], rows_v[b][q],
                                 sems[b])

        def finish_chunk(c, b):
            for q in range(4):
                pltpu.make_async_copy(table_hbm.at[idx_v[b].at[q]],
                                      rows_v[b][q], sems[b]).wait()

            @pl.when(c >= 2)
            def _drain_out():
                pltpu.make_async_copy(
                    out_v[b], out_hbm.at[pl.ds(base_w + (c - 2) * CH, CH)],
                    semo[b]).wait()

            @pl.loop(0, CH // L)
            def _grp(g):
                sl = pl.ds(g * L, L)
                a0 = w_v[b][0, sl]
                a1 = w_v[b][1, sl]
                b0 = w_v[b][2, sl]
                b1 = w_v[b][3, sl]
                for ll in range(L):
                    i = g * L + ll
                    lane = jnp.full((L,), ll, jnp.int32)
                    a0s = a0.at[lane].get(mode="promise_in_bounds")
                    a1s = a1.at[lane].get(mode="promise_in_bounds")
                    b0s = b0.at[lane].get(mode="promise_in_bounds")
                    b1s = b1.at[lane].get(mode="promise_in_bounds")
                    for j in range(cg):
                        cs = pl.ds(j * L, L)
                        t0 = (a0s * rows_v[b][0][i, cs]
                              + a1s * rows_v[b][1][i, cs])
                        t1 = (a0s * rows_v[b][2][i, cs]
                              + a1s * rows_v[b][3][i, cs])
                        out_v[b][i, cs] = b0s * t0 + b1s * t1

            pltpu.async_copy(out_v[b], out_hbm.at[pl.ds(base_w + c * CH, CH)],
                             semo[b])

        stage_chunk(0, 0)

        @pl.loop(0, nchunk, step=2)
        def _chunk(k):
            for b in range(2):
                c = k + b

                @pl.when(c + 1 < nchunk)
                def _prefetch():
                    stage_chunk(c + 1, (b + 1) % 2)

                finish_chunk(c, b)

        for b in range(2):
            cl = nchunk - 2 + b
            pltpu.make_async_copy(
                out_v[b], out_hbm.at[pl.ds(base_w + cl * CH, CH)],
                semo[b]).wait()

    return grid_sample_sc


def kernel(inp, grid):
    N, C, H, W = inp.shape
    _, Ho, Wo, _ = grid.shape
    table = inp.transpose(0, 2, 3, 1).reshape(N * H * W, C)
    gx = grid[..., 0].reshape(-1)
    gy = grid[..., 1].reshape(-1)
    sc = _make_sc_kernel(N, C, H, W, Ho, Wo)
    out_rows = sc(gx, gy, table)
    return out_rows.reshape(N, Ho, Wo, C).transpose(0, 3, 1, 2)
